# Initial kernel scaffold; baseline (speedup 1.0000x reference)
import jax
import jax.numpy as jnp
from jax.experimental import pallas as pl


def kernel(feature, typeEmbed, tableEmbed, columnEmbed, opEmbed, joinEmbed, Wf, bf, Wf2, bf2, Ws, bs, Wh, bh, Wp, bp):
    raise NotImplementedError("write your pallas kernel here")


# speedup vs baseline: 4.0180x; 4.0180x over previous
import jax
import jax.numpy as jnp
from jax.experimental import pallas as pl

_BT = 256
_ES = 64
_BIN = 50


def _leaky(x):
    return jnp.where(x >= 0, x, 0.01 * x)


def _block(x_ref, typeE_ref, tableE_ref, colE_ref, opE_ref, joinE_ref,
           Wf_ref, bf_ref, Wf2_ref, bf2_ref, Ws_ref, bs_ref,
           Wh_ref, bh_ref, Wp_ref, bp_ref, out_ref):
    f32 = jnp.float32
    dot = lambda a, b: jax.lax.dot(a, b, preferred_element_type=f32)

    head = x_ref[:, 0:256]
    typeId = head[:, 0:1]
    joinId = head[:, 1:2]
    tbl = head[:, 164:165]

    Wp = Wp_ref[...]

    Tt = dot(typeE_ref[...], Wp[0:64, :])
    Tj = dot(joinE_ref[...], Wp[137:201, :])
    Ttb = dot(tableE_ref[...], Wp[201:265, :])
    T50 = jnp.concatenate([Tt, Tj, Ttb], axis=0)
    lane50 = jax.lax.broadcasted_iota(jnp.int32, (1, 50), 1)
    oh50 = ((lane50 == typeId.astype(jnp.int32)).astype(f32)
            + (lane50 == joinId.astype(jnp.int32) + 20).astype(f32)
            + (lane50 == tbl.astype(jnp.int32) + 30).astype(f32))
    g = dot(oh50, T50)

    Wf = Wf_ref[...]
    ct = dot(colE_ref[...], Wf[0:64, :])
    ot = dot(opE_ref[...], Wf[64:72, :])
    wfv = Wf[72:73, :]
    W41 = jnp.concatenate([ct, ot, wfv], axis=0)
    bf = bf_ref[...]
    Wf2 = Wf2_ref[...]
    bf2 = bf2_ref[...]
    lane30 = jax.lax.broadcasted_iota(jnp.int32, (1, 30), 1)
    lane10 = jax.lax.broadcasted_iota(jnp.int32, (1, 10), 1)

    m0 = head[:, 11:12]
    m1 = head[:, 12:13]
    m2 = head[:, 13:14]
    msum = m0 + m1 + m2
    num = jnp.maximum(msum, 1.0)

    facc = jnp.zeros((_BT, 73), f32)
    for j in range(3):
        colj = head[:, 2 + j:3 + j]
        opj = head[:, 5 + j:6 + j]
        valj = head[:, 8 + j:9 + j]
        mj = head[:, 11 + j:12 + j]
        oh = jnp.concatenate([(lane30 == colj.astype(jnp.int32)).astype(f32),
                              (lane10 == opj.astype(jnp.int32)).astype(f32),
                              valj], axis=1)
        h1 = _leaky(dot(oh, W41) + bf)
        h2 = _leaky(dot(h1, Wf2) + bf2)
        facc = facc + mj * h2
    filterEmbed = facc / num

    Wh = Wh_ref[...]
    r150 = jax.lax.broadcasted_iota(jnp.int32, (150, 1), 0)
    c50 = jax.lax.broadcasted_iota(jnp.int32, (1, 50), 1)
    rep = dot((r150 // 3 == c50).astype(f32), Wh)
    mod3 = r150 % 3
    Whh = jnp.concatenate([jnp.where(mod3 == 0, rep, 0.0),
                           jnp.where(mod3 == 1, rep, 0.0),
                           jnp.where(mod3 == 2, rep, 0.0)], axis=1)
    hx = head[:, 14:164]
    hist3 = dot(hx, Whh)
    bh = bh_ref[...]
    histEmb = (m0 * hist3[:, 0:64] + m1 * hist3[:, 64:128]
               + m2 * hist3[:, 128:192] + msum * bh) / num

    samp = dot(x_ref[:, 165:1165], Ws_ref[...])
    tablePart = samp + bs_ref[...]

    pre = (g
           + dot(filterEmbed, Wp[64:137, :])
           + dot(tablePart, Wp[201:265, :])
           + dot(histEmb, Wp[265:329, :])
           + bp_ref[...])
    out_ref[...] = _leaky(pre)


def kernel(feature, typeEmbed, tableEmbed, columnEmbed, opEmbed, joinEmbed,
           Wf, bf, Wf2, bf2, Ws, bs, Wh, bh, Wp, bp):
    B = feature.shape[0]
    grid = B // _BT
    weights = [typeEmbed, tableEmbed, columnEmbed, opEmbed, joinEmbed,
               Wf, bf.reshape(1, -1), Wf2, bf2.reshape(1, -1),
               Ws, bs.reshape(1, -1), Wh, bh.reshape(1, -1),
               Wp, bp.reshape(1, -1)]

    def wspec(w):
        return pl.BlockSpec(w.shape, lambda i: (0, 0))

    return pl.pallas_call(
        _block,
        grid=(grid,),
        in_specs=[pl.BlockSpec((_BT, feature.shape[1]), lambda i: (i, 0))]
                 + [wspec(w) for w in weights],
        out_specs=pl.BlockSpec((_BT, _ES), lambda i: (i, 0)),
        out_shape=jax.ShapeDtypeStruct((B, _ES), jnp.float32),
    )(feature, *weights)


# speedup vs baseline: 4.7982x; 1.1942x over previous
import jax
import jax.numpy as jnp
from jax.experimental import pallas as pl
from jax.experimental.pallas import tpu as pltpu

_BT = 1024
_ES = 64
_BIN = 50


def _leaky(x):
    return jnp.where(x >= 0, x, 0.01 * x)


def _block(x_ref, typeE_ref, tableE_ref, colE_ref, opE_ref, joinE_ref,
           Wf_ref, bf_ref, Wf2_ref, bf2_ref, Ws_ref, bs_ref,
           Wh_ref, bh_ref, Wp_ref, bp_ref, out_ref):
    f32 = jnp.float32
    dot = lambda a, b: jax.lax.dot(a, b, preferred_element_type=f32)

    head = x_ref[:, 0:256]
    typeId = head[:, 0:1]
    joinId = head[:, 1:2]
    tbl = head[:, 164:165]

    Wp = Wp_ref[...]

    Tt = dot(typeE_ref[...], Wp[0:64, :])
    Tj = dot(joinE_ref[...], Wp[137:201, :])
    Ttb = dot(tableE_ref[...], Wp[201:265, :])
    T50 = jnp.concatenate([Tt, Tj, Ttb], axis=0)
    lane50 = jax.lax.broadcasted_iota(jnp.int32, (1, 50), 1)
    oh50 = ((lane50 == typeId.astype(jnp.int32)).astype(f32)
            + (lane50 == joinId.astype(jnp.int32) + 20).astype(f32)
            + (lane50 == tbl.astype(jnp.int32) + 30).astype(f32))
    g = dot(oh50, T50)

    Wf = Wf_ref[...]
    ct = dot(colE_ref[...], Wf[0:64, :])
    ot = dot(opE_ref[...], Wf[64:72, :])
    wfv = Wf[72:73, :]
    W41 = jnp.concatenate([ct, ot, wfv], axis=0)
    bf = bf_ref[...]
    Wf2 = Wf2_ref[...]
    bf2 = bf2_ref[...]
    lane30 = jax.lax.broadcasted_iota(jnp.int32, (1, 30), 1)
    lane10 = jax.lax.broadcasted_iota(jnp.int32, (1, 10), 1)

    m0 = head[:, 11:12]
    m1 = head[:, 12:13]
    m2 = head[:, 13:14]
    msum = m0 + m1 + m2
    num = jnp.maximum(msum, 1.0)

    facc = jnp.zeros((_BT, 73), f32)
    for j in range(3):
        colj = head[:, 2 + j:3 + j]
        opj = head[:, 5 + j:6 + j]
        valj = head[:, 8 + j:9 + j]
        mj = head[:, 11 + j:12 + j]
        oh = jnp.concatenate([(lane30 == colj.astype(jnp.int32)).astype(f32),
                              (lane10 == opj.astype(jnp.int32)).astype(f32),
                              valj], axis=1)
        h1 = _leaky(dot(oh, W41) + bf)
        h2 = _leaky(dot(h1, Wf2) + bf2)
        facc = facc + mj * h2
    filterEmbed = facc / num

    Wh = Wh_ref[...]
    r150 = jax.lax.broadcasted_iota(jnp.int32, (150, 1), 0)
    c50 = jax.lax.broadcasted_iota(jnp.int32, (1, 50), 1)
    rep = dot((r150 // 3 == c50).astype(f32), Wh)
    mod3 = r150 % 3
    Whh = jnp.concatenate([jnp.where(mod3 == 0, rep, 0.0),
                           jnp.where(mod3 == 1, rep, 0.0),
                           jnp.where(mod3 == 2, rep, 0.0)], axis=1)
    hx = head[:, 14:164]
    hist3 = dot(hx, Whh)
    bh = bh_ref[...]
    histEmb = (m0 * hist3[:, 0:64] + m1 * hist3[:, 64:128]
               + m2 * hist3[:, 128:192] + msum * bh) / num

    samp = dot(x_ref[:, 128:1165], Ws_ref[...][0:1037, :])
    tablePart = samp + bs_ref[...]

    pre = (g
           + dot(filterEmbed, Wp[64:137, :])
           + dot(tablePart, Wp[201:265, :])
           + dot(histEmb, Wp[265:329, :])
           + bp_ref[...])
    out_ref[...] = _leaky(pre)


def kernel(feature, typeEmbed, tableEmbed, columnEmbed, opEmbed, joinEmbed,
           Wf, bf, Wf2, bf2, Ws, bs, Wh, bh, Wp, bp):
    B = feature.shape[0]
    grid = B // _BT
    Ws_shift = jnp.concatenate(
        [jnp.zeros((37, _ES), Ws.dtype), Ws, jnp.zeros((3, _ES), Ws.dtype)], axis=0)
    weights = [typeEmbed, tableEmbed, columnEmbed, opEmbed, joinEmbed,
               Wf, bf.reshape(1, -1), Wf2, bf2.reshape(1, -1),
               Ws_shift, bs.reshape(1, -1), Wh, bh.reshape(1, -1),
               Wp, bp.reshape(1, -1)]

    def wspec(w):
        return pl.BlockSpec(w.shape, lambda i: (0, 0))

    return pl.pallas_call(
        _block,
        grid=(grid,),
        in_specs=[pl.BlockSpec((_BT, feature.shape[1]), lambda i: (i, 0))]
                 + [wspec(w) for w in weights],
        out_specs=pl.BlockSpec((_BT, _ES), lambda i: (i, 0)),
        out_shape=jax.ShapeDtypeStruct((B, _ES), jnp.float32),
        compiler_params=pltpu.CompilerParams(
            dimension_semantics=("parallel",)),
    )(feature, *weights)


# speedup vs baseline: 5.0090x; 1.0439x over previous
import jax
import jax.numpy as jnp
from jax.experimental import pallas as pl
from jax.experimental.pallas import tpu as pltpu

_BT = 1024
_ES = 64


def _leaky(x):
    return jnp.where(x >= 0, x, 0.01 * x)


def _dot(a, b):
    return jax.lax.dot(a, b, preferred_element_type=jnp.float32)


def _prep(typeE_ref, tableE_ref, colE_ref, opE_ref, joinE_ref,
          Wf_ref, Wh_ref, Wp_ref, T64_ref, W48_ref, Whh_ref):
    f32 = jnp.float32
    Wp = Wp_ref[...]
    Tt = _dot(typeE_ref[...], Wp[0:64, :])
    Tj = _dot(joinE_ref[...], Wp[137:201, :])
    Ttb = _dot(tableE_ref[...], Wp[201:265, :])
    T64_ref[...] = jnp.concatenate(
        [Tt, Tj, Ttb, jnp.zeros((14, 64), f32)], axis=0)

    Wf = Wf_ref[...]
    ct = _dot(colE_ref[...], Wf[0:64, :])
    ot = _dot(opE_ref[...], Wf[64:72, :])
    W48_ref[...] = jnp.concatenate(
        [ct, ot, Wf[72:73, :], jnp.zeros((7, 73), f32)], axis=0)

    Wh = Wh_ref[...]
    r150 = jax.lax.broadcasted_iota(jnp.int32, (150, 1), 0)
    c50 = jax.lax.broadcasted_iota(jnp.int32, (1, 50), 1)
    rep = _dot((r150 // 3 == c50).astype(f32), Wh)
    mod3 = r150 % 3
    strided = jnp.concatenate([jnp.where(mod3 == 0, rep, 0.0),
                               jnp.where(mod3 == 1, rep, 0.0),
                               jnp.where(mod3 == 2, rep, 0.0)], axis=1)
    Whh_ref[...] = jnp.concatenate(
        [jnp.zeros((14, 192), f32), strided, jnp.zeros((92, 192), f32)], axis=0)


def _block(x_ref, T64_ref, W48_ref, Wf2_ref, bf_ref, bf2_ref, Whh_ref, bh_ref,
           Ws_ref, bs_ref, Wpf_ref, Wptb_ref, Wph_ref, bp_ref, out_ref):
    f32 = jnp.float32
    i32 = jnp.int32

    l64 = jax.lax.broadcasted_iota(i32, (1, 64), 1)
    typeId = x_ref[:, 0:1].astype(i32)
    joinId = x_ref[:, 1:2].astype(i32)
    tbl = x_ref[:, 164:165].astype(i32)
    oh64 = ((l64 == typeId).astype(f32) + (l64 == joinId + 20).astype(f32)
            + (l64 == tbl + 30).astype(f32))
    acc = _dot(oh64, T64_ref[...])

    l48 = jax.lax.broadcasted_iota(i32, (1, 48), 1)
    W48 = W48_ref[...]
    Wf2 = Wf2_ref[...]
    bf = bf_ref[...]
    bf2 = bf2_ref[...]
    m = []
    facc = jnp.zeros((_BT, 73), f32)
    for j in range(3):
        coli = x_ref[:, 2 + j:3 + j].astype(i32)
        opi = x_ref[:, 5 + j:6 + j].astype(i32)
        valv = x_ref[:, 8 + j:9 + j]
        mj = x_ref[:, 11 + j:12 + j]
        m.append(mj)
        oh48 = ((l48 == coli).astype(f32) + (l48 == opi + 30).astype(f32)
                + valv * (l48 == 40).astype(f32))
        h1 = _leaky(_dot(oh48, W48) + bf)
        h2 = _leaky(_dot(h1, Wf2) + bf2)
        facc = facc + mj * h2
    msum = m[0] + m[1] + m[2]
    rnum = 1.0 / jnp.maximum(msum, 1.0)

    hist3 = _dot(x_ref[:, 0:256], Whh_ref[...])
    histEmb = (m[0] * hist3[:, 0:64] + m[1] * hist3[:, 64:128]
               + m[2] * hist3[:, 128:192] + msum * bh_ref[...]) * rnum

    samp = _dot(x_ref[:, 128:1165], Ws_ref[...][0:1037, :])

    pre = (acc
           + _dot(facc * rnum, Wpf_ref[...])
           + _dot(samp + bs_ref[...], Wptb_ref[...])
           + _dot(histEmb, Wph_ref[...])
           + bp_ref[...])
    out_ref[...] = _leaky(pre)


def _full(w):
    return pl.BlockSpec(w.shape, lambda i: tuple(0 for _ in w.shape))


def kernel(feature, typeEmbed, tableEmbed, columnEmbed, opEmbed, joinEmbed,
           Wf, bf, Wf2, bf2, Ws, bs, Wh, bh, Wp, bp):
    B = feature.shape[0]
    f32 = jnp.float32

    T64, W48, Whh = pl.pallas_call(
        _prep,
        out_shape=(jax.ShapeDtypeStruct((64, 64), f32),
                   jax.ShapeDtypeStruct((48, 73), f32),
                   jax.ShapeDtypeStruct((256, 192), f32)),
    )(typeEmbed, tableEmbed, columnEmbed, opEmbed, joinEmbed, Wf, Wh, Wp)

    Ws_shift = jnp.concatenate(
        [jnp.zeros((37, _ES), f32), Ws, jnp.zeros((3, _ES), f32)], axis=0)
    weights = [T64, W48, Wf2, bf.reshape(1, -1), bf2.reshape(1, -1),
               Whh, bh.reshape(1, -1), Ws_shift, bs.reshape(1, -1),
               Wp[64:137, :], Wp[201:265, :], Wp[265:329, :],
               bp.reshape(1, -1)]

    grid = B // _BT
    return pl.pallas_call(
        _block,
        grid=(grid,),
        in_specs=[pl.BlockSpec((_BT, feature.shape[1]), lambda i: (i, 0))]
                 + [_full(w) for w in weights],
        out_specs=pl.BlockSpec((_BT, _ES), lambda i: (i, 0)),
        out_shape=jax.ShapeDtypeStruct((B, _ES), f32),
        compiler_params=pltpu.CompilerParams(
            dimension_semantics=("parallel",)),
    )(feature, *weights)
